# Initial kernel scaffold; baseline (speedup 1.0000x reference)
import jax
import jax.numpy as jnp
from jax.experimental import pallas as pl
from jax.experimental.pallas import tpu as pltpu

B, S, D = 1, 2048, 1024
H = 4096
E = 8
K = 2
T = B * S

R = 256
NB = 24
NH = 4
HC = H // NH
NSLOT = NB * R


def _gelu(x):
    return x * 0.5 * (1.0 + jax.lax.erf(x * 0.7071067811865476))


def _router_kernel(x_ref, w_ref, b_ref, d0_ref, d1_ref, p0_ref, p1_ref,
                   meta_ref):
    x = x_ref[...]
    w = w_ref[...]
    logits = jax.lax.dot_general(
        x, w, (((1,), (1,)), ((), ())),
        precision=jax.lax.Precision.HIGHEST,
        preferred_element_type=jnp.float32)
    logits = logits + b_ref[...]

    e_iota = jax.lax.broadcasted_iota(jnp.int32, (T, E), 1)
    m1 = jnp.max(logits, axis=1, keepdims=True)
    cand1 = jnp.where(logits >= m1, e_iota, E + 1)
    i1 = jnp.min(cand1, axis=1, keepdims=True)
    sel0 = (e_iota == i1)
    masked = jnp.where(sel0, -jnp.inf, logits)
    m2 = jnp.max(masked, axis=1, keepdims=True)
    cand2 = jnp.where(masked >= m2, e_iota, E + 1)
    i2 = jnp.min(cand2, axis=1, keepdims=True)
    sel1 = (e_iota == i2)

    r = jnp.exp(m2 - m1)
    p0 = 1.0 / (1.0 + r)
    p1 = 1.0 - p0

    sel0_i = sel0.astype(jnp.int32)
    sel1_i = sel1.astype(jnp.int32)
    sel = sel0_i + sel1_i

    arr = sel
    shift = 1
    while shift < T:
        z = jnp.zeros((shift, E), jnp.int32)
        arr = arr + jnp.concatenate([z, arr[: T - shift, :]], axis=0)
        shift *= 2
    pos_excl = arr - sel
    counts = arr[T - 1:T, :]

    nb_row = (counts + (R - 1)) // R
    tri = (jax.lax.broadcasted_iota(jnp.int32, (E, E), 0) <
           jax.lax.broadcasted_iota(jnp.int32, (E, E), 1)).astype(jnp.float32)
    start_row = jax.lax.dot_general(
        nb_row.astype(jnp.float32), tri, (((1,), (0,)), ((), ())),
        preferred_element_type=jnp.float32).astype(jnp.int32)
    slot_base = start_row * R

    dest = slot_base + pos_excl
    d0_ref[...] = jnp.sum(jnp.where(sel0, dest, 0), axis=1, keepdims=True)
    d1_ref[...] = jnp.sum(jnp.where(sel1, dest, 0), axis=1, keepdims=True)
    p0_ref[...] = p0
    p1_ref[...] = p1

    nbu = jnp.sum(nb_row, axis=1, keepdims=True)
    ones_col = jnp.ones((E, 1), jnp.float32)
    outer = jax.lax.dot_general(
        ones_col, start_row.astype(jnp.float32), (((1,), (0,)), ((), ())),
        preferred_element_type=jnp.float32)
    diag = (jax.lax.broadcasted_iota(jnp.int32, (E, E), 0) ==
            jax.lax.broadcasted_iota(jnp.int32, (E, E), 1))
    start_col = jnp.sum(jnp.where(diag, outer, 0.0), axis=1,
                        keepdims=True).astype(jnp.int32)
    blk_iota = jax.lax.broadcasted_iota(jnp.int32, (E, NB), 1)
    le = (start_col <= blk_iota).astype(jnp.int32)
    eb_raw = jnp.sum(le, axis=0, keepdims=True) - 1
    nb_iota = jax.lax.broadcasted_iota(jnp.int32, (1, NB), 1)
    eb_last = jnp.sum(jnp.where(nb_iota == nbu - 1, eb_raw, 0), axis=1,
                      keepdims=True)
    eb = jnp.where(nb_iota < nbu, eb_raw, eb_last)

    meta_ref[:, :NB] = eb
    meta_ref[:, NB:] = jnp.broadcast_to(nbu, (1, 8))


def _run_router(x, router_w, bias_row):
    return pl.pallas_call(
        _router_kernel,
        out_shape=[
            jax.ShapeDtypeStruct((T, 1), jnp.int32),
            jax.ShapeDtypeStruct((T, 1), jnp.int32),
            jax.ShapeDtypeStruct((T, 1), jnp.float32),
            jax.ShapeDtypeStruct((T, 1), jnp.float32),
            jax.ShapeDtypeStruct((1, NB + 8), jnp.int32),
        ],
    )(x, router_w, bias_row)


def _dispatch_kernel(d0r_ref, d1r_ref, x_ref, xs_ref):
    b = pl.program_id(0)
    slots = jax.lax.broadcasted_iota(jnp.int32, (R, T), 0) + b * R
    sT = (d0r_ref[...] == slots) | (d1r_ref[...] == slots)
    sT = sT.astype(jnp.bfloat16)
    xs = jax.lax.dot_general(
        sT, x_ref[...], (((1,), (0,)), ((), ())),
        preferred_element_type=jnp.float32)
    xs_ref[...] = xs.astype(jnp.bfloat16)


def _run_dispatch(d0r, d1r, xb16):
    return pl.pallas_call(
        _dispatch_kernel,
        grid=(NB,),
        in_specs=[
            pl.BlockSpec((1, T), lambda b: (0, 0)),
            pl.BlockSpec((1, T), lambda b: (0, 0)),
            pl.BlockSpec((T, D), lambda b: (0, 0)),
        ],
        out_specs=pl.BlockSpec((R, D), lambda b: (b, 0)),
        out_shape=jax.ShapeDtypeStruct((NSLOT, D), jnp.bfloat16),
    )(d0r, d1r, xb16)


def _ffn_kernel(eb_ref, nbu_ref, xs_ref, fc1w_ref, fc1b_ref, fc2w_ref,
                fc2b_ref, d0_ref, d1_ref, p0_ref, p1_ref, o_ref, yb_ref):
    h = pl.program_id(0)
    b = pl.program_id(1)
    nbu = nbu_ref[0]

    @pl.when(jnp.logical_and(h == 0, b == 0))
    def _():
        o_ref[...] = jnp.zeros_like(o_ref)

    @pl.when(b < nbu)
    def _():
        xs = xs_ref[...]
        fc1w = fc1w_ref[0]
        hh = jax.lax.dot_general(
            xs, fc1w.astype(jnp.bfloat16), (((1,), (1,)), ((), ())),
            preferred_element_type=jnp.float32)
        hh = _gelu(hh + fc1b_ref[...])
        fc2w = fc2w_ref[0]
        part = jax.lax.dot_general(
            hh.astype(jnp.bfloat16), fc2w.astype(jnp.bfloat16),
            (((1,), (1,)), ((), ())),
            preferred_element_type=jnp.float32)
        row = pl.ds(b * R, R)

        @pl.when(h == 0)
        def _():
            yb_ref[row, :] = part + fc2b_ref[...]

        @pl.when(h > 0)
        def _():
            yb_ref[row, :] = yb_ref[row, :] + part

        @pl.when(h == NH - 1)
        def _():
            slots = jax.lax.broadcasted_iota(jnp.int32, (T, R), 1) + b * R
            c = (p0_ref[...] * (d0_ref[...] == slots) +
                 p1_ref[...] * (d1_ref[...] == slots))
            o_ref[...] += jax.lax.dot_general(
                c.astype(jnp.bfloat16), yb_ref[row, :].astype(jnp.bfloat16),
                (((1,), (0,)), ((), ())),
                preferred_element_type=jnp.float32)


def _run_ffn(eb, nbu, xs, fc1_w, fc1_b, fc2_w, fc2_b, d0, d1, p0, p1):
    grid_spec = pltpu.PrefetchScalarGridSpec(
        num_scalar_prefetch=2,
        grid=(NH, NB),
        in_specs=[
            pl.BlockSpec((R, D), lambda h, b, eb, nbu: (b, 0)),
            pl.BlockSpec((1, HC, D), lambda h, b, eb, nbu: (eb[b], h, 0)),
            pl.BlockSpec((1, HC), lambda h, b, eb, nbu: (eb[b], h)),
            pl.BlockSpec((1, D, HC), lambda h, b, eb, nbu: (eb[b], 0, h)),
            pl.BlockSpec((1, D), lambda h, b, eb, nbu: (eb[b], 0)),
            pl.BlockSpec((T, 1), lambda h, b, eb, nbu: (0, 0)),
            pl.BlockSpec((T, 1), lambda h, b, eb, nbu: (0, 0)),
            pl.BlockSpec((T, 1), lambda h, b, eb, nbu: (0, 0)),
            pl.BlockSpec((T, 1), lambda h, b, eb, nbu: (0, 0)),
        ],
        out_specs=pl.BlockSpec((T, D), lambda h, b, eb, nbu: (0, 0)),
        scratch_shapes=[pltpu.VMEM((NSLOT, D), jnp.float32)],
    )
    return pl.pallas_call(
        _ffn_kernel,
        grid_spec=grid_spec,
        out_shape=jax.ShapeDtypeStruct((T, D), jnp.float32),
    )(eb, nbu, xs, fc1_w, fc1_b, fc2_w, fc2_b, d0, d1, p0, p1)


@jax.jit
def kernel(hidden_states, router_w, router_b, gate_bias, fc1_w, fc1_b,
           fc2_w, fc2_b):
    b, s, d = hidden_states.shape
    x = hidden_states.reshape(T, D)
    bias_row = (router_b + gate_bias).reshape(1, E)
    d0, d1, p0, p1, meta = _run_router(x, router_w, bias_row)
    eb = meta[0, :NB]
    nbu = meta[0, NB:NB + 1]
    d0r = d0.reshape(1, T)
    d1r = d1.reshape(1, T)
    xb16 = x.astype(jnp.bfloat16)
    xs = _run_dispatch(d0r, d1r, xb16)
    out = _run_ffn(eb, nbu, xs, fc1_w, fc1_b, fc2_w, fc2_b, d0, d1, p0, p1)
    return out.reshape(b, s, d)


# speedup vs baseline: 4.9509x; 4.9509x over previous
import jax
import jax.numpy as jnp
from jax.experimental import pallas as pl
from jax.experimental.pallas import tpu as pltpu

B, S, D = 1, 2048, 1024
H = 4096
E = 8
K = 2
T = B * S

R = 256
NB = 24
NH = 4
HC = H // NH
NSLOT = NB * R


def _gelu(x):
    return x * 0.5 * (1.0 + jax.lax.erf(x * 0.7071067811865476))


def _router_kernel(x_ref, w_ref, b_ref, d0_ref, d1_ref, p0_ref, p1_ref,
                   meta_ref):
    x = x_ref[...]
    w = w_ref[...]
    logits = jax.lax.dot_general(
        x.astype(jnp.bfloat16), w.astype(jnp.bfloat16), (((1,), (1,)), ((), ())),
        preferred_element_type=jnp.float32)
    logits = logits + b_ref[...]

    e_iota = jax.lax.broadcasted_iota(jnp.int32, (T, E), 1)
    m1 = jnp.max(logits, axis=1, keepdims=True)
    cand1 = jnp.where(logits >= m1, e_iota, E + 1)
    i1 = jnp.min(cand1, axis=1, keepdims=True)
    sel0 = (e_iota == i1)
    masked = jnp.where(sel0, -jnp.inf, logits)
    m2 = jnp.max(masked, axis=1, keepdims=True)
    cand2 = jnp.where(masked >= m2, e_iota, E + 1)
    i2 = jnp.min(cand2, axis=1, keepdims=True)
    sel1 = (e_iota == i2)

    r = jnp.exp(m2 - m1)
    p0 = 1.0 / (1.0 + r)
    p1 = 1.0 - p0

    sel0_i = sel0.astype(jnp.int32)
    sel1_i = sel1.astype(jnp.int32)
    sel = sel0_i + sel1_i

    arr = sel
    shift = 1
    while shift < T:
        z = jnp.zeros((shift, E), jnp.int32)
        arr = arr + jnp.concatenate([z, arr[: T - shift, :]], axis=0)
        shift *= 2
    pos_excl = arr - sel
    counts = arr[T - 1:T, :]

    nb_row = (counts + (R - 1)) // R
    tri = (jax.lax.broadcasted_iota(jnp.int32, (E, E), 0) <
           jax.lax.broadcasted_iota(jnp.int32, (E, E), 1)).astype(jnp.float32)
    start_row = jax.lax.dot_general(
        nb_row.astype(jnp.float32), tri, (((1,), (0,)), ((), ())),
        preferred_element_type=jnp.float32).astype(jnp.int32)
    slot_base = start_row * R

    dest = slot_base + pos_excl
    d0_ref[...] = jnp.sum(jnp.where(sel0, dest, 0), axis=1, keepdims=True)
    d1_ref[...] = jnp.sum(jnp.where(sel1, dest, 0), axis=1, keepdims=True)
    p0_ref[...] = p0
    p1_ref[...] = p1

    nbu = jnp.sum(nb_row, axis=1, keepdims=True)
    ones_col = jnp.ones((E, 1), jnp.float32)
    outer = jax.lax.dot_general(
        ones_col, start_row.astype(jnp.float32), (((1,), (0,)), ((), ())),
        preferred_element_type=jnp.float32)
    diag = (jax.lax.broadcasted_iota(jnp.int32, (E, E), 0) ==
            jax.lax.broadcasted_iota(jnp.int32, (E, E), 1))
    start_col = jnp.sum(jnp.where(diag, outer, 0.0), axis=1,
                        keepdims=True).astype(jnp.int32)
    blk_iota = jax.lax.broadcasted_iota(jnp.int32, (E, NB), 1)
    le = (start_col <= blk_iota).astype(jnp.int32)
    eb_raw = jnp.sum(le, axis=0, keepdims=True) - 1
    nb_iota = jax.lax.broadcasted_iota(jnp.int32, (1, NB), 1)
    eb_last = jnp.sum(jnp.where(nb_iota == nbu - 1, eb_raw, 0), axis=1,
                      keepdims=True)
    eb = jnp.where(nb_iota < nbu, eb_raw, eb_last)

    meta_ref[:, :NB] = eb
    meta_ref[:, NB:] = jnp.broadcast_to(nbu, (1, 8))


def _run_router(x, router_w, bias_row):
    return pl.pallas_call(
        _router_kernel,
        out_shape=[
            jax.ShapeDtypeStruct((T, 1), jnp.int32),
            jax.ShapeDtypeStruct((T, 1), jnp.int32),
            jax.ShapeDtypeStruct((T, 1), jnp.float32),
            jax.ShapeDtypeStruct((T, 1), jnp.float32),
            jax.ShapeDtypeStruct((1, NB + 8), jnp.int32),
        ],
    )(x, router_w, bias_row)


def _dispatch_kernel(d0r_ref, d1r_ref, x_ref, xs_ref):
    b = pl.program_id(0)
    slots = jax.lax.broadcasted_iota(jnp.int32, (R, T), 0) + b * R
    sT = (d0r_ref[...] == slots) | (d1r_ref[...] == slots)
    sT = sT.astype(jnp.bfloat16)
    xs = jax.lax.dot_general(
        sT, x_ref[...], (((1,), (0,)), ((), ())),
        preferred_element_type=jnp.float32)
    xs_ref[...] = xs.astype(jnp.bfloat16)


def _run_dispatch(d0r, d1r, xb16):
    return pl.pallas_call(
        _dispatch_kernel,
        grid=(NB,),
        in_specs=[
            pl.BlockSpec((1, T), lambda b: (0, 0)),
            pl.BlockSpec((1, T), lambda b: (0, 0)),
            pl.BlockSpec((T, D), lambda b: (0, 0)),
        ],
        out_specs=pl.BlockSpec((R, D), lambda b: (b, 0)),
        out_shape=jax.ShapeDtypeStruct((NSLOT, D), jnp.bfloat16),
    )(d0r, d1r, xb16)


def _ffn_kernel(eb_ref, nbu_ref, xs_ref, fc1w_ref, fc1b_ref, fc2w_ref,
                fc2b_ref, d0_ref, d1_ref, p0_ref, p1_ref, o_ref, yb_ref):
    h = pl.program_id(0)
    b = pl.program_id(1)
    nbu = nbu_ref[0]

    @pl.when(jnp.logical_and(h == 0, b == 0))
    def _():
        o_ref[...] = jnp.zeros_like(o_ref)

    @pl.when(b < nbu)
    def _():
        xs = xs_ref[...]
        fc1w = fc1w_ref[0]
        hh = jax.lax.dot_general(
            xs, fc1w.astype(jnp.bfloat16), (((1,), (1,)), ((), ())),
            preferred_element_type=jnp.float32)
        hh = _gelu(hh + fc1b_ref[0])
        fc2w = fc2w_ref[0]
        part = jax.lax.dot_general(
            hh.astype(jnp.bfloat16), fc2w.astype(jnp.bfloat16),
            (((1,), (1,)), ((), ())),
            preferred_element_type=jnp.float32)
        row = pl.ds(b * R, R)

        @pl.when(h == 0)
        def _():
            yb_ref[row, :] = part + fc2b_ref[0]

        @pl.when(h > 0)
        def _():
            yb_ref[row, :] = yb_ref[row, :] + part

        @pl.when(h == NH - 1)
        def _():
            slots = jax.lax.broadcasted_iota(jnp.int32, (T, R), 1) + b * R
            c = (p0_ref[...] * (d0_ref[...] == slots) +
                 p1_ref[...] * (d1_ref[...] == slots))
            o_ref[...] += jax.lax.dot_general(
                c.astype(jnp.bfloat16), yb_ref[row, :].astype(jnp.bfloat16),
                (((1,), (0,)), ((), ())),
                preferred_element_type=jnp.float32)


def _run_ffn(eb, nbu, xs, fc1_w, fc1_b, fc2_w, fc2_b, d0, d1, p0, p1):
    grid_spec = pltpu.PrefetchScalarGridSpec(
        num_scalar_prefetch=2,
        grid=(NH, NB),
        in_specs=[
            pl.BlockSpec((R, D), lambda h, b, eb, nbu: (b, 0)),
            pl.BlockSpec((1, HC, D), lambda h, b, eb, nbu: (eb[b], h, 0)),
            pl.BlockSpec((1, 1, HC), lambda h, b, eb, nbu: (eb[b] * NH + h, 0, 0)),
            pl.BlockSpec((1, D, HC), lambda h, b, eb, nbu: (eb[b], 0, h)),
            pl.BlockSpec((1, 1, D), lambda h, b, eb, nbu: (eb[b], 0, 0)),
            pl.BlockSpec((T, 1), lambda h, b, eb, nbu: (0, 0)),
            pl.BlockSpec((T, 1), lambda h, b, eb, nbu: (0, 0)),
            pl.BlockSpec((T, 1), lambda h, b, eb, nbu: (0, 0)),
            pl.BlockSpec((T, 1), lambda h, b, eb, nbu: (0, 0)),
        ],
        out_specs=pl.BlockSpec((T, D), lambda h, b, eb, nbu: (0, 0)),
        scratch_shapes=[pltpu.VMEM((NSLOT, D), jnp.float32)],
    )
    return pl.pallas_call(
        _ffn_kernel,
        grid_spec=grid_spec,
        out_shape=jax.ShapeDtypeStruct((T, D), jnp.float32),
    )(eb, nbu, xs, fc1_w, fc1_b.reshape(E * NH, 1, HC), fc2_w,
      fc2_b.reshape(E, 1, D), d0, d1, p0, p1)


@jax.jit
def kernel(hidden_states, router_w, router_b, gate_bias, fc1_w, fc1_b,
           fc2_w, fc2_b):
    b, s, d = hidden_states.shape
    x = hidden_states.reshape(T, D)
    bias_row = (router_b + gate_bias).reshape(1, E)
    d0, d1, p0, p1, meta = _run_router(x, router_w, bias_row)
    eb = meta[0, :NB]
    nbu = meta[0, NB:NB + 1]
    d0r = d0.reshape(1, T)
    d1r = d1.reshape(1, T)
    xb16 = x.astype(jnp.bfloat16)
    xs = _run_dispatch(d0r, d1r, xb16)
    out = _run_ffn(eb, nbu, xs, fc1_w, fc1_b, fc2_w, fc2_b, d0, d1, p0, p1)
    return out.reshape(b, s, d)
